# baseline (speedup 1.0000x reference)
import functools

import jax
import jax.numpy as jnp
from jax import lax
from jax.experimental import pallas as pl
from jax.experimental.pallas import tpu as pltpu
from jax.experimental.pallas import tpu_sc as plsc

N_FACTORS = 16
NC = 2
NS = 16
NW = NC * NS
L = 16
CHUNK = 128


def _mf_body(iu_hbm, ii_hbm, u_hbm, v_hbm, w_hbm, b_hbm, out_hbm,
             iu_v, ii_v, urows, vrows, w_v, b_v, out_v, sem):
    b_per_w = iu_v.shape[0]
    wid = lax.axis_index("s") * NC + lax.axis_index("c")
    base = wid * b_per_w

    pltpu.sync_copy(iu_hbm.at[pl.ds(base, b_per_w)], iu_v)
    pltpu.sync_copy(ii_hbm.at[pl.ds(base, b_per_w)], ii_v)
    pltpu.sync_copy(w_hbm, w_v)
    pltpu.sync_copy(b_hbm, b_v)

    copies = []
    for c in range(0, b_per_w, CHUNK):
        copies.append(pltpu.make_async_copy(
            u_hbm.at[iu_v.at[pl.ds(c, CHUNK)]], urows.at[pl.ds(c, CHUNK)], sem))
        copies.append(pltpu.make_async_copy(
            v_hbm.at[ii_v.at[pl.ds(c, CHUNK)]], vrows.at[pl.ds(c, CHUNK)], sem))
    for cp in copies:
        cp.start()
    for cp in copies:
        cp.wait()

    w_bcast = [w_v[f, :] for f in range(N_FACTORS)]
    bvec = b_v[...]
    iota = lax.iota(jnp.int32, L)

    def blk_body(blk, carry):
        rows = blk * L + iota
        acc = bvec
        for f in range(N_FACTORS):
            fs = jnp.full((L,), f, jnp.int32)
            gu = plsc.load_gather(urows, [rows, fs])
            gv = plsc.load_gather(vrows, [rows, fs])
            acc = acc + gu * gv * w_bcast[f]
        out_v[pl.ds(blk * L, L)] = acc
        return carry

    lax.fori_loop(0, b_per_w // L, blk_body, 0)

    pltpu.sync_copy(out_v, out_hbm.at[pl.ds(base, b_per_w)])


@jax.jit
def kernel(idx_users, idx_items, user_emb_mf, item_emb_mf, W_out, b_out):
    B = idx_users.shape[0]
    b_per_w = B // NW
    mesh = plsc.VectorSubcoreMesh(core_axis_name="c", subcore_axis_name="s",
                                  num_cores=NC, num_subcores=NS)
    k = pl.kernel(
        _mf_body,
        out_type=jax.ShapeDtypeStruct((B,), jnp.float32),
        mesh=mesh,
        scratch_types=[
            pltpu.VMEM((b_per_w,), jnp.int32),
            pltpu.VMEM((b_per_w,), jnp.int32),
            pltpu.VMEM((b_per_w, N_FACTORS), jnp.float32),
            pltpu.VMEM((b_per_w, N_FACTORS), jnp.float32),
            pltpu.VMEM((N_FACTORS, L), jnp.float32),
            pltpu.VMEM((N_FACTORS,), jnp.float32),
            pltpu.VMEM((b_per_w,), jnp.float32),
            pltpu.SemaphoreType.DMA,
        ],
        compiler_params=pltpu.CompilerParams(
            needs_layout_passes=False, use_tc_tiling_on_sc=False),
    )
    w16 = jnp.broadcast_to(
        W_out.reshape((N_FACTORS, 1)).astype(jnp.float32), (N_FACTORS, L))
    b16 = jnp.broadcast_to(b_out.reshape(()), (N_FACTORS,)).astype(jnp.float32)
    return k(idx_users.astype(jnp.int32), idx_items.astype(jnp.int32),
             user_emb_mf, item_emb_mf, w16, b16)


# speedup vs baseline: 5.7658x; 5.7658x over previous
import jax
import jax.numpy as jnp
from jax import lax
from jax.experimental import pallas as pl
from jax.experimental.pallas import tpu as pltpu
from jax.experimental.pallas import tpu_sc as plsc

N_FACTORS = 16
NC = 2
NS = 16
NW = NC * NS
L = 16
CH = 16


def _mf_body(iu_hbm, ii_hbm, ut_hbm, vt_hbm, w_hbm, b_hbm, out_hbm,
             iu_v, ii_v, ublk, vblk, w_v, b_v, out_v, sem_a, sem_b):
    b_per_w = iu_v.shape[0]
    nch = b_per_w // CH
    wid = lax.axis_index("s") * NC + lax.axis_index("c")
    base = pl.multiple_of(wid * b_per_w, b_per_w)

    pltpu.sync_copy(iu_hbm.at[pl.ds(base, b_per_w)], iu_v)
    pltpu.sync_copy(ii_hbm.at[pl.ds(base, b_per_w)], ii_v)
    pltpu.sync_copy(w_hbm, w_v)
    pltpu.sync_copy(b_hbm, b_v)

    w_bcast = [w_v[f, :] for f in range(N_FACTORS)]
    bvec = b_v[...]
    iota = lax.iota(jnp.int32, L)

    def issue(c, half, sem, p):
        uvec = iu_v[pl.ds(c * CH, CH)]
        vvec = ii_v[pl.ds(c * CH, CH)]
        for j in range(CH):
            ub = pl.multiple_of(
                lax.shift_left(lax.shift_right_logical(uvec[j], 7), 7), 128)
            pltpu.make_async_copy(
                ut_hbm.at[pl.ds(8 * p, 8), pl.ds(ub, 128)],
                ublk.at[half * CH + j], sem).start()
            vb = pl.multiple_of(
                lax.shift_left(lax.shift_right_logical(vvec[j], 7), 7), 128)
            pltpu.make_async_copy(
                vt_hbm.at[pl.ds(8 * p, 8), pl.ds(vb, 128)],
                vblk.at[half * CH + j], sem).start()

    def drain(half, sem):
        for j in range(CH):
            pltpu.make_async_copy(
                ut_hbm.at[pl.ds(0, 8), pl.ds(0, 128)],
                ublk.at[half * CH + j], sem).wait()
            pltpu.make_async_copy(
                ut_hbm.at[pl.ds(0, 8), pl.ds(0, 128)],
                vblk.at[half * CH + j], sem).wait()

    def compute(c, half, p):
        uvec = iu_v[pl.ds(c * CH, CH)]
        vvec = ii_v[pl.ds(c * CH, CH)]
        umod = lax.bitwise_and(uvec, jnp.int32(127))
        vmod = lax.bitwise_and(vvec, jnp.int32(127))
        slots = half * CH + iota
        if p == 0:
            acc = bvec
        else:
            acc = out_v[pl.ds(c * CH, CH)]
        for f in range(8):
            fs = jnp.full((L,), f, jnp.int32)
            gu = plsc.load_gather(ublk, [slots, fs, umod])
            gv = plsc.load_gather(vblk, [slots, fs, vmod])
            acc = acc + gu * gv * w_bcast[8 * p + f]
        out_v[pl.ds(c * CH, CH)] = acc

    for p in range(2):
        issue(0, 0, sem_a, p)

        def pair_body(k, carry, p=p):
            issue(2 * k + 1, 1, sem_b, p)
            drain(0, sem_a)
            compute(2 * k, 0, p)
            issue(jnp.minimum(2 * k + 2, nch - 1), 0, sem_a, p)
            drain(1, sem_b)
            compute(2 * k + 1, 1, p)
            return carry

        lax.fori_loop(0, nch // 2, pair_body, 0)
        drain(0, sem_a)

    pltpu.sync_copy(out_v, out_hbm.at[pl.ds(base, b_per_w)])


@jax.jit
def kernel(idx_users, idx_items, user_emb_mf, item_emb_mf, W_out, b_out):
    B = idx_users.shape[0]
    b_per_w = B // NW
    mesh = plsc.VectorSubcoreMesh(core_axis_name="c", subcore_axis_name="s",
                                  num_cores=NC, num_subcores=NS)
    k = pl.kernel(
        _mf_body,
        out_type=jax.ShapeDtypeStruct((B,), jnp.float32),
        mesh=mesh,
        scratch_types=[
            pltpu.VMEM((b_per_w,), jnp.int32),
            pltpu.VMEM((b_per_w,), jnp.int32),
            pltpu.VMEM((2 * CH, 8, 128), jnp.float32),
            pltpu.VMEM((2 * CH, 8, 128), jnp.float32),
            pltpu.VMEM((N_FACTORS, L), jnp.float32),
            pltpu.VMEM((L,), jnp.float32),
            pltpu.VMEM((b_per_w,), jnp.float32),
            pltpu.SemaphoreType.DMA,
            pltpu.SemaphoreType.DMA,
        ],
        compiler_params=pltpu.CompilerParams(
            needs_layout_passes=False, use_tc_tiling_on_sc=True),
    )
    w16 = jnp.broadcast_to(
        W_out.reshape((N_FACTORS, 1)).astype(jnp.float32), (N_FACTORS, L))
    b16 = jnp.broadcast_to(b_out.reshape(()).astype(jnp.float32), (L,))
    return k(idx_users.astype(jnp.int32), idx_items.astype(jnp.int32),
             user_emb_mf.T, item_emb_mf.T, w16, b16)
